# baseline (speedup 1.0000x reference)
import functools

import jax
import jax.numpy as jnp
from jax import lax
from jax.experimental import pallas as pl
from jax.experimental.pallas import tpu as pltpu
from jax.experimental.pallas import tpu_sc as plsc

_NUM_FIELDS = 26
_FIELD_DIM = 100000
_BATCH = 4096
_LANES = 16
_NC = 2
_NS = 16
_NW = _NC * _NS
_BPW = _BATCH // _NW
_NCHUNK = 2
_CB = _BPW // _NCHUNK
_CIDX = _NUM_FIELDS * _CB
_NIDX = _NUM_FIELDS * _BPW


def _make_sc_kernel():
    mesh = plsc.VectorSubcoreMesh(core_axis_name="c", subcore_axis_name="s")

    @functools.partial(
        pl.kernel,
        mesh=mesh,
        out_type=jax.ShapeDtypeStruct((_BATCH,), jnp.float32),
        scratch_types=[
            pltpu.VMEM((_NUM_FIELDS, _BPW), jnp.int32),
            pltpu.VMEM((_NIDX,), jnp.int32),
            pltpu.VMEM((_NIDX,), jnp.float32),
            pltpu.VMEM((_BPW,), jnp.float32),
            pltpu.VMEM((_LANES,), jnp.float32),
            [pltpu.SemaphoreType.DMA] * _NCHUNK,
        ],
    )
    def k(xt_hbm, table_hbm, bias_hbm, out_hbm,
          xb_v, idx_v, rows_v, ob_v, bias_v, sems):
        wid = lax.axis_index("s") * _NC + lax.axis_index("c")
        base = wid * _BPW
        pltpu.sync_copy(xt_hbm.at[:, pl.ds(base, _BPW)], xb_v)
        pltpu.sync_copy(bias_hbm, bias_v)
        bias_vec = bias_v[...]
        nq = _CB // _LANES

        copies = []
        for chunk in range(_NCHUNK):
            cbase = chunk * _CIDX

            def build(f, carry, chunk=chunk, cbase=cbase):
                off = f * _FIELD_DIM
                for q in range(nq):
                    idx_v[pl.ds(cbase + f * _CB + q * _LANES, _LANES)] = (
                        xb_v[f, pl.ds(chunk * _CB + q * _LANES, _LANES)] + off)
                return carry

            lax.fori_loop(0, _NUM_FIELDS, build, 0)
            copies.append(pltpu.async_copy(
                table_hbm.at[idx_v.at[pl.ds(cbase, _CIDX)]],
                rows_v.at[pl.ds(cbase, _CIDX)], sems[chunk]))

        for chunk in range(_NCHUNK):
            cbase = chunk * _CIDX
            copies[chunk].wait()

            def accum(f, accs, cbase=cbase):
                return tuple(
                    accs[q] + rows_v[pl.ds(cbase + f * _CB + q * _LANES,
                                           _LANES)]
                    for q in range(nq)
                )

            accs = lax.fori_loop(0, _NUM_FIELDS, accum, (bias_vec,) * nq)
            for q in range(nq):
                ob_v[pl.ds(chunk * _CB + q * _LANES, _LANES)] = (
                    1.0 / (1.0 + jnp.exp(-accs[q])))
        pltpu.sync_copy(ob_v, out_hbm.at[pl.ds(base, _BPW)])

    return k


_sc_kernel = _make_sc_kernel()


@jax.jit
def kernel(x, table, bias):
    bias16 = jnp.broadcast_to(bias, (_LANES,))
    return _sc_kernel(x.T, table.reshape(-1), bias16)


# speedup vs baseline: 1.0011x; 1.0011x over previous
import functools

import jax
import jax.numpy as jnp
from jax import lax
from jax.experimental import pallas as pl
from jax.experimental.pallas import tpu as pltpu
from jax.experimental.pallas import tpu_sc as plsc

_NUM_FIELDS = 26
_FIELD_DIM = 100000
_BATCH = 4096
_LANES = 16
_NC = 2
_NS = 16
_NW = _NC * _NS
_BPW = _BATCH // _NW
_NCHUNK = 8
_CB = _BPW // _NCHUNK
_CIDX = _NUM_FIELDS * _CB
_NIDX = _NUM_FIELDS * _BPW


def _make_sc_kernel():
    mesh = plsc.VectorSubcoreMesh(core_axis_name="c", subcore_axis_name="s")

    @functools.partial(
        pl.kernel,
        mesh=mesh,
        out_type=jax.ShapeDtypeStruct((_BATCH,), jnp.float32),
        scratch_types=[
            pltpu.VMEM((_NUM_FIELDS, _BPW), jnp.int32),
            pltpu.VMEM((_NIDX,), jnp.int32),
            pltpu.VMEM((_NIDX,), jnp.float32),
            pltpu.VMEM((_BPW,), jnp.float32),
            pltpu.VMEM((_LANES,), jnp.float32),
            [pltpu.SemaphoreType.DMA] * _NCHUNK,
        ],
    )
    def k(xt_hbm, table_hbm, bias_hbm, out_hbm,
          xb_v, idx_v, rows_v, ob_v, bias_v, sems):
        wid = lax.axis_index("s") * _NC + lax.axis_index("c")
        base = wid * _BPW
        pltpu.sync_copy(xt_hbm.at[:, pl.ds(base, _BPW)], xb_v)
        pltpu.sync_copy(bias_hbm, bias_v)
        bias_vec = bias_v[...]
        nq = _CB // _LANES

        copies = []
        for chunk in range(_NCHUNK):
            cbase = chunk * _CIDX

            def build(f, carry, chunk=chunk, cbase=cbase):
                off = f * _FIELD_DIM
                for q in range(nq):
                    idx_v[pl.ds(cbase + f * _CB + q * _LANES, _LANES)] = (
                        xb_v[f, pl.ds(chunk * _CB + q * _LANES, _LANES)] + off)
                return carry

            lax.fori_loop(0, _NUM_FIELDS, build, 0)
            copies.append(pltpu.async_copy(
                table_hbm.at[idx_v.at[pl.ds(cbase, _CIDX)]],
                rows_v.at[pl.ds(cbase, _CIDX)], sems[chunk]))

        for chunk in range(_NCHUNK):
            cbase = chunk * _CIDX
            copies[chunk].wait()

            def accum(f, accs, cbase=cbase):
                return tuple(
                    accs[q] + rows_v[pl.ds(cbase + f * _CB + q * _LANES,
                                           _LANES)]
                    for q in range(nq)
                )

            accs = lax.fori_loop(0, _NUM_FIELDS, accum, (bias_vec,) * nq)
            for q in range(nq):
                ob_v[pl.ds(chunk * _CB + q * _LANES, _LANES)] = (
                    1.0 / (1.0 + jnp.exp(-accs[q])))
        pltpu.sync_copy(ob_v, out_hbm.at[pl.ds(base, _BPW)])

    return k


_sc_kernel = _make_sc_kernel()


@jax.jit
def kernel(x, table, bias):
    bias16 = jnp.broadcast_to(bias, (_LANES,))
    return _sc_kernel(x.T, table.reshape(-1), bias16)


# speedup vs baseline: 1.0029x; 1.0018x over previous
import functools

import jax
import jax.numpy as jnp
from jax import lax
from jax.experimental import pallas as pl
from jax.experimental.pallas import tpu as pltpu
from jax.experimental.pallas import tpu_sc as plsc

_NUM_FIELDS = 26
_FIELD_DIM = 100000
_BATCH = 4096
_LANES = 16
_NC = 2
_NS = 16
_NW = _NC * _NS
_BPW = _BATCH // _NW
_NCHUNK = 4
_CB = _BPW // _NCHUNK
_CIDX = _NUM_FIELDS * _CB
_NIDX = _NUM_FIELDS * _BPW


def _make_sc_kernel():
    mesh = plsc.VectorSubcoreMesh(core_axis_name="c", subcore_axis_name="s")

    @functools.partial(
        pl.kernel,
        mesh=mesh,
        out_type=jax.ShapeDtypeStruct((_BATCH,), jnp.float32),
        scratch_types=[
            pltpu.VMEM((_NUM_FIELDS, _BPW), jnp.int32),
            pltpu.VMEM((_NIDX,), jnp.int32),
            pltpu.VMEM((_NIDX,), jnp.float32),
            pltpu.VMEM((_BPW,), jnp.float32),
            pltpu.VMEM((_LANES,), jnp.float32),
            [pltpu.SemaphoreType.DMA] * _NCHUNK,
        ],
    )
    def k(xt_hbm, table_hbm, bias_hbm, out_hbm,
          xb_v, idx_v, rows_v, ob_v, bias_v, sems):
        wid = lax.axis_index("s") * _NC + lax.axis_index("c")
        base = wid * _BPW
        pltpu.sync_copy(xt_hbm.at[:, pl.ds(base, _BPW)], xb_v)
        pltpu.sync_copy(bias_hbm, bias_v)
        bias_vec = bias_v[...]
        nq = _CB // _LANES

        copies = []
        for chunk in range(_NCHUNK):
            cbase = chunk * _CIDX

            def build(f, carry, chunk=chunk, cbase=cbase):
                off = f * _FIELD_DIM
                for q in range(nq):
                    idx_v[pl.ds(cbase + f * _CB + q * _LANES, _LANES)] = (
                        xb_v[f, pl.ds(chunk * _CB + q * _LANES, _LANES)] + off)
                return carry

            lax.fori_loop(0, _NUM_FIELDS, build, 0)
            copies.append(pltpu.async_copy(
                table_hbm.at[idx_v.at[pl.ds(cbase, _CIDX)]],
                rows_v.at[pl.ds(cbase, _CIDX)], sems[chunk]))

        for chunk in range(_NCHUNK):
            cbase = chunk * _CIDX
            copies[chunk].wait()

            def accum(f, accs, cbase=cbase):
                return tuple(
                    accs[q] + rows_v[pl.ds(cbase + f * _CB + q * _LANES,
                                           _LANES)]
                    for q in range(nq)
                )

            accs = lax.fori_loop(0, _NUM_FIELDS, accum, (bias_vec,) * nq)
            for q in range(nq):
                ob_v[pl.ds(chunk * _CB + q * _LANES, _LANES)] = (
                    1.0 / (1.0 + jnp.exp(-accs[q])))
        pltpu.sync_copy(ob_v, out_hbm.at[pl.ds(base, _BPW)])

    return k


_sc_kernel = _make_sc_kernel()


@jax.jit
def kernel(x, table, bias):
    bias16 = jnp.broadcast_to(bias, (_LANES,))
    return _sc_kernel(x.T, table.reshape(-1), bias16)
